# baseline (speedup 1.0000x reference)
import functools

import jax
import jax.numpy as jnp
from jax import lax
from jax.experimental import pallas as pl
from jax.experimental.pallas import tpu as pltpu
from jax.experimental.pallas import tpu_sc as plsc

N = 16384
K = 10
WIDTH = 64
VOCAB = 1001
B = N * K
D_OUT = K * WIDTH

TAB_ROWS = 504
TAB_MINOR = 128

NC = 2
NS = 16
NW = NC * NS
N_PER_W = N // NW
B_PER_W = B // NW
CH_S = 32
NCH = N_PER_W // CH_S
NSG = CH_S // 16
W_UNROLL = 8
NBUF = 2


def _gather_kernel(table_hbm, idx_hbm, out_hbm, tab_v, idx_v, bufs, wsems):
    wid = lax.axis_index("s") * NC + lax.axis_index("c")
    base = wid * B_PER_W
    n_base = wid * N_PER_W
    pltpu.sync_copy(table_hbm, tab_v)
    pltpu.sync_copy(idx_hbm.at[pl.ds(base, B_PER_W)], idx_v)

    lanes = lax.iota(jnp.int32, 16)

    def assemble(c, buf):
        for sg in range(NSG):
            i_vec = sg * 16 + lanes
            pos0 = (c * CH_S + sg * 16) * K
            lanes_k = lanes * K

            def slot(k, _):
                v_vec = plsc.load_gather(idx_v, [lanes_k + (pos0 + k)])
                r_vec = v_vec >> 1
                cb_vec = (v_vec & 1) * WIDTH
                jb = k * WIDTH

                def wblk(wb, _):
                    for u in range(W_UNROLL):
                        w = wb * W_UNROLL + u
                        x = plsc.load_gather(tab_v, [r_vec, cb_vec + w])
                        j_vec = jnp.full((16,), jb + w, jnp.int32)
                        plsc.store_scatter(buf, [i_vec, j_vec], x)
                    return _

                lax.fori_loop(0, WIDTH // W_UNROLL, wblk, None)
                return _

            lax.fori_loop(0, K, slot, None)

    whandles = [None] * NBUF
    for c in range(NCH):
        b = c % NBUF
        if whandles[b] is not None:
            whandles[b].wait()
        assemble(c, bufs[b])
        whandles[b] = pltpu.async_copy(
            bufs[b], out_hbm.at[pl.ds(n_base + c * CH_S, CH_S), :], wsems[b]
        )
    for b in range(NBUF):
        whandles[b].wait()


@jax.jit
def _lookup(uuid_values_flat, table_pad):
    mesh = plsc.VectorSubcoreMesh(core_axis_name="c", subcore_axis_name="s")
    k = functools.partial(
        pl.kernel,
        mesh=mesh,
        out_type=jax.ShapeDtypeStruct((N, D_OUT), jnp.float32),
        scratch_types=[
            pltpu.VMEM((TAB_ROWS, TAB_MINOR), jnp.float32),
            pltpu.VMEM((B_PER_W,), jnp.int32),
            [pltpu.VMEM((CH_S, D_OUT), jnp.float32) for _ in range(NBUF)],
            [pltpu.SemaphoreType.DMA for _ in range(NBUF)],
        ],
        compiler_params=pltpu.CompilerParams(use_tc_tiling_on_sc=False, needs_layout_passes=False),
    )(_gather_kernel)
    return k(table_pad, uuid_values_flat)


def kernel(partname_indices, pos_values, uuid_values, uuid_embedding):
    flat_tab = uuid_embedding.reshape(-1)
    pad = TAB_ROWS * TAB_MINOR - flat_tab.shape[0]
    table_pad = jnp.pad(flat_tab, (0, pad)).reshape(TAB_ROWS, TAB_MINOR)
    return _lookup(uuid_values.reshape(-1), table_pad)


# speedup vs baseline: 4.7898x; 4.7898x over previous
import functools

import jax
import jax.numpy as jnp
from jax import lax
from jax.experimental import pallas as pl
from jax.experimental.pallas import tpu as pltpu
from jax.experimental.pallas import tpu_sc as plsc

N = 16384
K = 10
WIDTH = 64
B = N * K

NC = 2
NS = 16
NW = NC * NS
NSPLIT = 1
BS = B // NSPLIT
B_PER_W = BS // NW
VOCAB = 1001
CH = 320
NCH = B_PER_W // CH
NBUF = 2


def _gather_kernel(table_hbm, idx_hbm, out_hbm, tab_v, idx_v, bufs, gsems, wsems):
    sid = lax.axis_index("s")
    wid = sid * NC + lax.axis_index("c")
    base = wid * B_PER_W

    @pl.when(sid == 0)
    def _stage_table():
        pltpu.sync_copy(table_hbm, tab_v)

    pltpu.sync_copy(idx_hbm.at[pl.ds(base, B_PER_W)], idx_v)
    plsc.subcore_barrier()

    def start_gather(c):
        b = c % NBUF
        return pltpu.async_copy(
            tab_v.at[idx_v.at[pl.ds(c * CH, CH)]], bufs[b], gsems[b]
        )

    def start_write(c):
        b = c % NBUF
        return pltpu.async_copy(
            bufs[b], out_hbm.at[pl.ds(base + c * CH, CH)], wsems[b]
        )

    ghandles = [None] * NBUF
    whandles = [None] * NBUF
    for c in range(NCH + NBUF - 1):
        if c < NCH:
            b = c % NBUF
            if whandles[b] is not None:
                whandles[b].wait()
            ghandles[b] = start_gather(c)
        d = c - (NBUF - 1)
        if d >= 0:
            db = d % NBUF
            ghandles[db].wait()
            whandles[db] = start_write(d)
    for b in range(NBUF):
        if whandles[b] is not None:
            whandles[b].wait()


@jax.jit
def _lookup(uuid_values_flat, uuid_embedding):
    mesh = plsc.VectorSubcoreMesh(core_axis_name="c", subcore_axis_name="s")
    k = functools.partial(
        pl.kernel,
        mesh=mesh,
        out_type=jax.ShapeDtypeStruct((BS, WIDTH), jnp.float32),
        scratch_types=[
            pltpu.VMEM_SHARED((VOCAB, WIDTH), jnp.float32),
            pltpu.VMEM((B_PER_W,), jnp.int32),
            [pltpu.VMEM((CH, WIDTH), jnp.float32) for _ in range(NBUF)],
            [pltpu.SemaphoreType.DMA for _ in range(NBUF)],
            [pltpu.SemaphoreType.DMA for _ in range(NBUF)],
        ],
        compiler_params=pltpu.CompilerParams(use_tc_tiling_on_sc=False),
    )(_gather_kernel)
    parts = [
        k(uuid_embedding, lax.slice(uuid_values_flat, (s * BS,), ((s + 1) * BS,)))
        for s in range(NSPLIT)
    ]
    return jnp.concatenate(parts, axis=0)


def kernel(partname_indices, pos_values, uuid_values, uuid_embedding):
    flat = _lookup(uuid_values.reshape(-1), uuid_embedding)
    return flat.reshape(N, K * WIDTH)


# speedup vs baseline: 4.8261x; 1.0076x over previous
import functools

import jax
import jax.numpy as jnp
from jax import lax
from jax.experimental import pallas as pl
from jax.experimental.pallas import tpu as pltpu
from jax.experimental.pallas import tpu_sc as plsc

N = 16384
K = 10
WIDTH = 64
B = N * K

NC = 2
NS = 16
NW = NC * NS
NSPLIT = 1
BS = B // NSPLIT
B_PER_W = BS // NW
VOCAB = 1001
CH = 320
NCH = B_PER_W // CH
NBUF = 4


def _gather_kernel(table_hbm, idx_hbm, out_hbm, tab_v, idx_v, bufs, gsems, wsems):
    sid = lax.axis_index("s")
    wid = sid * NC + lax.axis_index("c")
    base = wid * B_PER_W

    @pl.when(sid == 0)
    def _stage_table():
        pltpu.sync_copy(table_hbm, tab_v)

    pltpu.sync_copy(idx_hbm.at[pl.ds(base, B_PER_W)], idx_v)
    plsc.subcore_barrier()

    def start_gather(c):
        b = c % NBUF
        return pltpu.async_copy(
            tab_v.at[idx_v.at[pl.ds(c * CH, CH)]], bufs[b], gsems[b]
        )

    def start_write(c):
        b = c % NBUF
        return pltpu.async_copy(
            bufs[b], out_hbm.at[pl.ds(base + c * CH, CH)], wsems[b]
        )

    ghandles = [None] * NBUF
    whandles = [None] * NBUF
    for c in range(NCH + NBUF - 1):
        if c < NCH:
            b = c % NBUF
            if whandles[b] is not None:
                whandles[b].wait()
            ghandles[b] = start_gather(c)
        d = c - (NBUF - 1)
        if d >= 0:
            db = d % NBUF
            ghandles[db].wait()
            whandles[db] = start_write(d)
    for b in range(NBUF):
        if whandles[b] is not None:
            whandles[b].wait()


@jax.jit
def _lookup(uuid_values_flat, uuid_embedding):
    mesh = plsc.VectorSubcoreMesh(core_axis_name="c", subcore_axis_name="s")
    k = functools.partial(
        pl.kernel,
        mesh=mesh,
        out_type=jax.ShapeDtypeStruct((BS, WIDTH), jnp.float32),
        scratch_types=[
            pltpu.VMEM_SHARED((VOCAB, WIDTH), jnp.float32),
            pltpu.VMEM((B_PER_W,), jnp.int32),
            [pltpu.VMEM((CH, WIDTH), jnp.float32) for _ in range(NBUF)],
            [pltpu.SemaphoreType.DMA for _ in range(NBUF)],
            [pltpu.SemaphoreType.DMA for _ in range(NBUF)],
        ],
        compiler_params=pltpu.CompilerParams(use_tc_tiling_on_sc=False),
    )(_gather_kernel)
    parts = [
        k(uuid_embedding, lax.slice(uuid_values_flat, (s * BS,), ((s + 1) * BS,)))
        for s in range(NSPLIT)
    ]
    return jnp.concatenate(parts, axis=0)


def kernel(partname_indices, pos_values, uuid_values, uuid_embedding):
    flat = _lookup(uuid_values.reshape(-1), uuid_embedding)
    return flat.reshape(N, K * WIDTH)
